# baseline (speedup 1.0000x reference)
import functools

import numpy as np
import jax
import jax.numpy as jnp
from jax import lax
from jax.experimental import pallas as pl
from jax.experimental.pallas import tpu as pltpu
from jax.experimental.pallas import tpu_sc as plsc

SCALE = 1.0
L = 16
F = 2
LOG2_T = 19
T = 2 ** LOG2_T
MASK = T - 1
N_MIN = 16
B_GROW = float(np.exp(np.log(2048 * SCALE / N_MIN) / (L - 1)))
N = 262144

P1 = np.int32(np.uint32(2654435761).astype(np.int64) - 2**32)
P2 = np.int32(805459861)

LEVELS = []
for _l in range(L):
    _s = N_MIN * (B_GROW ** _l) - 1.0
    _res = int(np.ceil(_s)) + 1
    LEVELS.append((_s, _res, _res ** 3 <= T))

NC = 2
NS = 16
NW = NC * NS
PER_W = N // NW
CHUNK = 512
NCHUNK = PER_W // CHUNK
NGRP = CHUNK // 16


@functools.cache
def _encode_kernel():
    mesh = plsc.VectorSubcoreMesh(core_axis_name="c", subcore_axis_name="s")

    @functools.partial(
        pl.kernel,
        mesh=mesh,
        out_type=jax.ShapeDtypeStruct((2 * L, N), jnp.float32),
        scratch_types=[
            pltpu.VMEM((3, CHUNK), jnp.float32),
            pltpu.VMEM((2 * 8 * CHUNK,), jnp.int32),
            pltpu.VMEM((8, CHUNK), jnp.float32),
            pltpu.VMEM((2 * 8 * CHUNK,), jnp.float32),
            pltpu.VMEM((2 * L, CHUNK), jnp.float32),
            pltpu.SemaphoreType.DMA,
        ],
    )
    def encode(xn_hbm, table_hbm, h_hbm, xyz_v, idx_v, w_v, rows_v, h_v, sem):
        wid = lax.axis_index("s") * NC + lax.axis_index("c")
        base0 = wid * PER_W
        HALF = 8 * CHUNK

        def chunk_body(ci, carry):
            base = base0 + ci * CHUNK
            pltpu.sync_copy(xn_hbm.at[:, pl.ds(base, CHUNK)], xyz_v)

            for l in range(L):
                s, res, dense = LEVELS[l]
                off = l * T

                def p1_body(g, c2, s=s, res=res, dense=dense, off=off):
                    sl = pl.ds(g * 16, 16)
                    xs = xyz_v[0, sl]
                    ys = xyz_v[1, sl]
                    zs = xyz_v[2, sl]
                    px = xs * s + 0.5
                    py = ys * s + 0.5
                    pz = zs * s + 0.5
                    ix = px.astype(jnp.int32)
                    iy = py.astype(jnp.int32)
                    iz = pz.astype(jnp.int32)
                    fx = px - ix.astype(jnp.float32)
                    fy = py - iy.astype(jnp.float32)
                    fz = pz - iz.astype(jnp.float32)
                    mx = 1.0 - fx
                    my = 1.0 - fy
                    mz = 1.0 - fz
                    wyz = ((my * mz, my * fz), (fy * mz, fy * fz))
                    if dense:
                        dy0 = iy * (2 * res)
                        dy1 = dy0 + 2 * res
                        dz0 = iz * (2 * res * res) + 2 * off
                        dz1 = dz0 + 2 * res * res
                        eyz = ((dy0 + dz0, dy0 + dz1), (dy1 + dz0, dy1 + dz1))
                        ix0 = ix * 2
                        ix1 = ix0 + 2
                    else:
                        hy0 = iy * P1
                        hy1 = hy0 + P1
                        hz0 = iz * P2
                        hz1 = hz0 + P2
                        eyz = ((hy0 ^ hz0, hy0 ^ hz1), (hy1 ^ hz0, hy1 ^ hz1))
                        ix0 = ix
                        ix1 = ix + 1
                    for c in range(8):
                        cx, cy, cz = c & 1, (c >> 1) & 1, (c >> 2) & 1
                        xi = ix1 if cx else ix0
                        if dense:
                            e0 = xi + eyz[cy][cz]
                        else:
                            e0 = ((xi ^ eyz[cy][cz]) & MASK) * 2 + 2 * off
                        pos = c * CHUNK + g * 16
                        idx_v[pl.ds(pos, 16)] = e0
                        idx_v[pl.ds(HALF + pos, 16)] = e0 + 1
                        wx = fx if cx else mx
                        w_v[c, sl] = wx * wyz[cy][cz]
                    return c2

                lax.fori_loop(0, NGRP, p1_body, 0)

                pltpu.async_copy(table_hbm.at[idx_v], rows_v, sem).wait()

                def p2_body(g, c2, l=l):
                    sl = pl.ds(g * 16, 16)
                    acc0 = jnp.zeros((16,), jnp.float32)
                    acc1 = jnp.zeros((16,), jnp.float32)
                    for c in range(8):
                        pos = c * CHUNK + g * 16
                        f0 = rows_v[pl.ds(pos, 16)]
                        f1 = rows_v[pl.ds(HALF + pos, 16)]
                        w = w_v[c, sl]
                        acc0 = acc0 + w * f0
                        acc1 = acc1 + w * f1
                    h_v[2 * l, sl] = acc0
                    h_v[2 * l + 1, sl] = acc1
                    return c2

                lax.fori_loop(0, NGRP, p2_body, 0)

            pltpu.sync_copy(h_v, h_hbm.at[:, pl.ds(base, CHUNK)])
            return carry

        lax.fori_loop(0, NCHUNK, chunk_body, 0)

    return encode


BN = 2048


def _mlp_body(h_ref, w1_ref, b1_ref, w2_ref, b2_ref, o_ref):
    h = h_ref[...]
    z = jnp.dot(w1_ref[...], h, preferred_element_type=jnp.float32)
    z = z + b1_ref[...][:, None]
    z = jnp.maximum(z, 0.0) + jnp.log1p(jnp.exp(-jnp.abs(z)))
    o = jnp.dot(w2_ref[...], z, preferred_element_type=jnp.float32)
    o = o + b2_ref[...][:, None]
    o_ref[...] = (1.0 / (1.0 + jnp.exp(-o)))[0]


@functools.cache
def _mlp_kernel():
    return pl.pallas_call(
        _mlp_body,
        grid=(N // BN,),
        in_specs=[
            pl.BlockSpec((2 * L, BN), lambda i: (0, i)),
            pl.BlockSpec((64, 2 * L), lambda i: (0, 0)),
            pl.BlockSpec((64,), lambda i: (0,)),
            pl.BlockSpec((1, 64), lambda i: (0, 0)),
            pl.BlockSpec((1,), lambda i: (0,)),
        ],
        out_specs=pl.BlockSpec((BN,), lambda i: (i,)),
        out_shape=jax.ShapeDtypeStruct((N,), jnp.float32),
    )


def kernel(x, d, R_inv, hash_table, W1, b1, W2, b2, iso_color):
    xn = ((x @ R_inv.T) + SCALE) / (2.0 * SCALE)
    xn_t = xn.T
    table = hash_table.reshape(L * T * F)
    h = _encode_kernel()(xn_t, table)
    alphas = _mlp_kernel()(h, W1, b1, W2, b2)
    return (alphas, iso_color)
